# baseline (speedup 1.0000x reference)
import functools

import jax
import jax.numpy as jnp
from jax import lax
from jax.experimental import pallas as pl
from jax.experimental.pallas import tpu as pltpu
from jax.experimental.pallas import tpu_sc as plsc

HIDDEN = 128
LANES = 16
NREG = HIDDEN // LANES
NCORES = 2
NSUB = 16
NW = NCORES * NSUB
SEQ = 200
CH = 40
NCH = SEQ // CH
IDS_A = 0
PIDX = 64
AUXW = 128
NBUF = 3


def _body(aux, word, posb, consts, out, aux_v0, aux_v1, aux_v2,
          rows_0, rows_1, rows_2, posb_v, consts_v,
          sa0, sa1, sa2, sg0, sg1, sg2, so0, so1, so2):
    aux_bufs = (aux_v0, aux_v1, aux_v2)
    rows_bufs = (rows_0, rows_1, rows_2)
    sa = (sa0, sa1, sa2)
    sg = (sg0, sg1, sg2)
    so = (so0, so1, so2)
    cid = lax.axis_index("c")
    sid = lax.axis_index("s")
    wid = sid * NCORES + cid
    nchunks = out.shape[0] // (CH * NW)
    chunk0 = wid * nchunks

    pltpu.sync_copy(posb, posb_v)
    pltpu.sync_copy(consts, consts_v)

    gm = [consts_v[0, pl.ds(k * LANES, LANES)] for k in range(NREG)]
    bt = [consts_v[1, pl.ds(k * LANES, LANES)] for k in range(NREG)]

    def aux_copy(r, b):
        return pltpu.async_copy(
            aux.at[pl.ds((chunk0 + r) * AUXW, AUXW)], aux_bufs[b], sa[b])

    def gather_rows(b):
        idxr = aux_bufs[b]
        rows = rows_bufs[b]
        return pltpu.async_copy(word.at[idxr.at[pl.ds(IDS_A, CH)]],
                                rows, sg[b])

    def wait_gather(b):
        idxr = aux_bufs[b]
        rows = rows_bufs[b]
        pltpu.make_async_copy(word.at[idxr.at[pl.ds(IDS_A, CH)]],
                              rows, sg[b]).wait()

    def wait_out(b):
        pltpu.make_async_copy(rows_bufs[b], out.at[pl.ds(0, CH)], so[b]).wait()

    def compute_chunk(r, b):
        rows_v = rows_bufs[b]
        ttr = aux_bufs[b]

        @plsc.parallel_loop(0, CH, unroll=4)
        def tok_body(i):
            pi = ttr[pl.ds(PIDX + i, LANES)][0]
            xs = []
            for k in range(NREG):
                sl = pl.ds(k * LANES, LANES)
                x = rows_v[i, sl] + posb_v[pi, sl]
                xs.append(x)
            s01 = (xs[0] + xs[1]) + (xs[2] + xs[3])
            s23 = (xs[4] + xs[5]) + (xs[6] + xs[7])
            ssum = jnp.sum(s01 + s23)
            q01 = (xs[0] * xs[0] + xs[1] * xs[1]) + (xs[2] * xs[2] + xs[3] * xs[3])
            q23 = (xs[4] * xs[4] + xs[5] * xs[5]) + (xs[6] * xs[6] + xs[7] * xs[7])
            qsum = jnp.sum(q01 + q23)
            mean = ssum * (1.0 / HIDDEN)
            var = qsum * (1.0 / HIDDEN) - mean * mean
            veps = jnp.maximum(var, 0.0) + 1e-12
            v16 = lax.broadcast(veps, (LANES,))
            bits = plsc.bitcast(v16, jnp.int32)
            y = plsc.bitcast(jnp.int32(0x5F3759DF) - (bits >> 1), jnp.float32)
            nhalf = -0.5 * v16
            for _ in range(2):
                y = y * (1.5 + nhalf * y * y)
            for k in range(NREG):
                sl = pl.ds(k * LANES, LANES)
                g = gm[k] * y
                rows_v[i, sl] = (xs[k] - mean) * g + bt[k]

        pltpu.async_copy(rows_v, out.at[pl.ds((chunk0 + r) * CH, CH)], so[b])

    cpa0 = aux_copy(0, 0)
    aux_copy(1, 1)
    cpa0.wait()
    gather_rows(0)

    def loop_body(p, carry):
        for b in range(NBUF):
            r = p * NBUF + b
            bn = (b + 1) % NBUF
            bn2 = (b + 2) % NBUF

            @pl.when(r + 2 < nchunks)
            def _():
                aux_copy(r + 2, bn2)

            @pl.when(r + 1 < nchunks)
            def _():
                pltpu.make_async_copy(aux.at[pl.ds(0, AUXW)],
                                      aux_bufs[bn], sa[bn]).wait()

                @pl.when(r >= 2)
                def _():
                    wait_out(bn)

                gather_rows(bn)

            @pl.when(r < nchunks)
            def _():
                wait_gather(b)
                compute_chunk(r, b)
        return carry

    lax.fori_loop(0, pl.cdiv(nchunks, NBUF), loop_body, 0)

    for b in range(NBUF):
        wait_out(b)


def kernel(input_ids, token_type_ids, word_emb, pos_emb, type_emb, gamma, beta):
    B, L = input_ids.shape
    ids = input_ids.astype(jnp.int32)
    tt = token_type_ids.astype(jnp.int32)
    nchunk = B * NCH
    pidx = tt * SEQ + jnp.arange(L, dtype=jnp.int32)[None, :]
    aux = jnp.zeros((nchunk, AUXW), jnp.int32)
    aux = aux.at[:, IDS_A:IDS_A + CH].set(ids.reshape(nchunk, CH))
    aux = aux.at[:, PIDX:PIDX + CH].set(pidx.reshape(nchunk, CH))
    posb = jnp.concatenate(
        [pos_emb[:L] + type_emb[0][None, :], pos_emb[:L] + type_emb[1][None, :]])
    consts = jnp.stack([gamma, beta])

    mesh = plsc.VectorSubcoreMesh(core_axis_name="c", subcore_axis_name="s")
    run = functools.partial(
        pl.kernel,
        mesh=mesh,
        out_type=jax.ShapeDtypeStruct((B * L, HIDDEN), jnp.float32),
        compiler_params=pltpu.CompilerParams(needs_layout_passes=False),
        scratch_types=[
            pltpu.VMEM((AUXW,), jnp.int32),
            pltpu.VMEM((AUXW,), jnp.int32),
            pltpu.VMEM((AUXW,), jnp.int32),
            pltpu.VMEM((CH, HIDDEN), jnp.float32),
            pltpu.VMEM((CH, HIDDEN), jnp.float32),
            pltpu.VMEM((CH, HIDDEN), jnp.float32),
            pltpu.VMEM((2 * SEQ, HIDDEN), jnp.float32),
            pltpu.VMEM((2, HIDDEN), jnp.float32),
        ] + [pltpu.SemaphoreType.DMA] * 9,
    )(_body)
    out = run(aux.reshape(nchunk * AUXW), word_emb, posb, consts)
    return out.reshape(B, L, HIDDEN)


# speedup vs baseline: 1.2918x; 1.2918x over previous
import functools

import jax
import jax.numpy as jnp
from jax import lax
from jax.experimental import pallas as pl
from jax.experimental.pallas import tpu as pltpu
from jax.experimental.pallas import tpu_sc as plsc

HIDDEN = 128
LANES = 16
NREG = HIDDEN // LANES
NCORES = 2
NSUB = 16
NW = NCORES * NSUB
SEQ = 200
CH = 128
IDS_A = 0
PIDX = 128
AUXW = 384
NBUF = 3


def _body(aux, word, posb, consts, out, aux_v0, aux_v1, aux_v2,
          rows_0, rows_1, rows_2, posb_v, consts_v,
          sa0, sa1, sa2, sg0, sg1, sg2, so0, so1, so2):
    aux_bufs = (aux_v0, aux_v1, aux_v2)
    rows_bufs = (rows_0, rows_1, rows_2)
    sa = (sa0, sa1, sa2)
    sg = (sg0, sg1, sg2)
    so = (so0, so1, so2)
    cid = lax.axis_index("c")
    sid = lax.axis_index("s")
    wid = sid * NCORES + cid
    nchunks = out.shape[0] // (CH * NW)
    chunk0 = wid * nchunks

    pltpu.sync_copy(posb, posb_v)
    pltpu.sync_copy(consts, consts_v)

    gm = [consts_v[0, pl.ds(k * LANES, LANES)] for k in range(NREG)]
    bt = [consts_v[1, pl.ds(k * LANES, LANES)] for k in range(NREG)]

    def aux_copy(r, b):
        return pltpu.async_copy(
            aux.at[pl.ds((chunk0 + r) * AUXW, AUXW)], aux_bufs[b], sa[b])

    def gather_rows(b):
        idxr = aux_bufs[b]
        rows = rows_bufs[b]
        return pltpu.async_copy(word.at[idxr.at[pl.ds(IDS_A, CH)]],
                                rows, sg[b])

    def wait_gather(b):
        idxr = aux_bufs[b]
        rows = rows_bufs[b]
        pltpu.make_async_copy(word.at[idxr.at[pl.ds(IDS_A, CH)]],
                              rows, sg[b]).wait()

    def wait_out(b):
        pltpu.make_async_copy(rows_bufs[b], out.at[pl.ds(0, CH)], so[b]).wait()

    def compute_chunk(r, b):
        rows_v = rows_bufs[b]
        ttr = aux_bufs[b]

        @plsc.parallel_loop(0, CH, unroll=4)
        def tok_body(i):
            pi = ttr[pl.ds(PIDX + i, LANES)][0]
            xs = []
            for k in range(NREG):
                sl = pl.ds(k * LANES, LANES)
                x = rows_v[i, sl] + posb_v[pi, sl]
                xs.append(x)
            s01 = (xs[0] + xs[1]) + (xs[2] + xs[3])
            s23 = (xs[4] + xs[5]) + (xs[6] + xs[7])
            ssum = jnp.sum(s01 + s23)
            q01 = (xs[0] * xs[0] + xs[1] * xs[1]) + (xs[2] * xs[2] + xs[3] * xs[3])
            q23 = (xs[4] * xs[4] + xs[5] * xs[5]) + (xs[6] * xs[6] + xs[7] * xs[7])
            qsum = jnp.sum(q01 + q23)
            mean = ssum * (1.0 / HIDDEN)
            var = qsum * (1.0 / HIDDEN) - mean * mean
            veps = jnp.maximum(var, 0.0) + 1e-12
            v16 = lax.broadcast(veps, (LANES,))
            bits = plsc.bitcast(v16, jnp.int32)
            y = plsc.bitcast(jnp.int32(0x5F3759DF) - (bits >> 1), jnp.float32)
            nhalf = -0.5 * v16
            for _ in range(2):
                y = y * (1.5 + nhalf * y * y)
            for k in range(NREG):
                sl = pl.ds(k * LANES, LANES)
                g = gm[k] * y
                rows_v[i, sl] = (xs[k] - mean) * g + bt[k]

        pltpu.async_copy(rows_v, out.at[pl.ds((chunk0 + r) * CH, CH)], so[b])

    cpa0 = aux_copy(0, 0)
    aux_copy(1, 1)
    cpa0.wait()
    gather_rows(0)

    def loop_body(p, carry):
        for b in range(NBUF):
            r = p * NBUF + b
            bn = (b + 1) % NBUF
            bn2 = (b + 2) % NBUF

            @pl.when(r + 2 < nchunks)
            def _():
                aux_copy(r + 2, bn2)

            @pl.when(r + 1 < nchunks)
            def _():
                pltpu.make_async_copy(aux.at[pl.ds(0, AUXW)],
                                      aux_bufs[bn], sa[bn]).wait()

                @pl.when(r >= 2)
                def _():
                    wait_out(bn)

                gather_rows(bn)

            @pl.when(r < nchunks)
            def _():
                wait_gather(b)
                compute_chunk(r, b)
        return carry

    lax.fori_loop(0, pl.cdiv(nchunks, NBUF), loop_body, 0)

    for b in range(NBUF):
        wait_out(b)


def kernel(input_ids, token_type_ids, word_emb, pos_emb, type_emb, gamma, beta):
    B, L = input_ids.shape
    ids = input_ids.astype(jnp.int32)
    tt = token_type_ids.astype(jnp.int32)
    nchunk = (B * L) // CH
    pidx = tt * SEQ + jnp.arange(L, dtype=jnp.int32)[None, :]
    aux = jnp.zeros((nchunk, AUXW), jnp.int32)
    aux = aux.at[:, IDS_A:IDS_A + CH].set(ids.reshape(nchunk, CH))
    aux = aux.at[:, PIDX:PIDX + CH].set(pidx.reshape(nchunk, CH))
    posb = jnp.concatenate(
        [pos_emb[:L] + type_emb[0][None, :], pos_emb[:L] + type_emb[1][None, :]])
    consts = jnp.stack([gamma, beta])

    mesh = plsc.VectorSubcoreMesh(core_axis_name="c", subcore_axis_name="s")
    run = functools.partial(
        pl.kernel,
        mesh=mesh,
        out_type=jax.ShapeDtypeStruct((B * L, HIDDEN), jnp.float32),
        compiler_params=pltpu.CompilerParams(needs_layout_passes=False),
        scratch_types=[
            pltpu.VMEM((AUXW,), jnp.int32),
            pltpu.VMEM((AUXW,), jnp.int32),
            pltpu.VMEM((AUXW,), jnp.int32),
            pltpu.VMEM((CH, HIDDEN), jnp.float32),
            pltpu.VMEM((CH, HIDDEN), jnp.float32),
            pltpu.VMEM((CH, HIDDEN), jnp.float32),
            pltpu.VMEM((2 * SEQ, HIDDEN), jnp.float32),
            pltpu.VMEM((2, HIDDEN), jnp.float32),
        ] + [pltpu.SemaphoreType.DMA] * 9,
    )(_body)
    out = run(aux.reshape(nchunk * AUXW), word_emb, posb, consts)
    return out.reshape(B, L, HIDDEN)


# speedup vs baseline: 1.3047x; 1.0100x over previous
import functools

import jax
import jax.numpy as jnp
from jax import lax
from jax.experimental import pallas as pl
from jax.experimental.pallas import tpu as pltpu
from jax.experimental.pallas import tpu_sc as plsc

HIDDEN = 128
LANES = 16
NREG = HIDDEN // LANES
NCORES = 2
NSUB = 16
NW = NCORES * NSUB
SEQ = 200
CH = 128
IDS_A = 0
PIDX = 128
AUXW = 384
NBUF = 4


def _body(aux, word, posb, consts, out, aux_v0, aux_v1, aux_v2, aux_v3,
          rows_0, rows_1, rows_2, rows_3, posb_v, consts_v,
          sa0, sa1, sa2, sa3, sg0, sg1, sg2, sg3, so0, so1, so2, so3):
    aux_bufs = (aux_v0, aux_v1, aux_v2, aux_v3)
    rows_bufs = (rows_0, rows_1, rows_2, rows_3)
    sa = (sa0, sa1, sa2, sa3)
    sg = (sg0, sg1, sg2, sg3)
    so = (so0, so1, so2, so3)
    cid = lax.axis_index("c")
    sid = lax.axis_index("s")
    wid = sid * NCORES + cid
    nchunks = out.shape[0] // (CH * NW)
    chunk0 = wid * nchunks

    pltpu.sync_copy(posb, posb_v)
    pltpu.sync_copy(consts, consts_v)

    gm = [consts_v[0, pl.ds(k * LANES, LANES)] for k in range(NREG)]
    bt = [consts_v[1, pl.ds(k * LANES, LANES)] for k in range(NREG)]

    def aux_copy(r, b):
        return pltpu.async_copy(
            aux.at[pl.ds((chunk0 + r) * AUXW, AUXW)], aux_bufs[b], sa[b])

    def gather_rows(b):
        idxr = aux_bufs[b]
        rows = rows_bufs[b]
        return pltpu.async_copy(word.at[idxr.at[pl.ds(IDS_A, CH)]],
                                rows, sg[b])

    def wait_gather(b):
        idxr = aux_bufs[b]
        rows = rows_bufs[b]
        pltpu.make_async_copy(word.at[idxr.at[pl.ds(IDS_A, CH)]],
                              rows, sg[b]).wait()

    def wait_out(b):
        pltpu.make_async_copy(rows_bufs[b], out.at[pl.ds(0, CH)], so[b]).wait()

    def compute_chunk(r, b):
        rows_v = rows_bufs[b]
        ttr = aux_bufs[b]

        @plsc.parallel_loop(0, CH, unroll=4)
        def tok_body(i):
            pi = ttr[pl.ds(PIDX + i, LANES)][0]
            xs = []
            for k in range(NREG):
                sl = pl.ds(k * LANES, LANES)
                x = rows_v[i, sl] + posb_v[pi, sl]
                xs.append(x)
            s01 = (xs[0] + xs[1]) + (xs[2] + xs[3])
            s23 = (xs[4] + xs[5]) + (xs[6] + xs[7])
            ssum = jnp.sum(s01 + s23)
            q01 = (xs[0] * xs[0] + xs[1] * xs[1]) + (xs[2] * xs[2] + xs[3] * xs[3])
            q23 = (xs[4] * xs[4] + xs[5] * xs[5]) + (xs[6] * xs[6] + xs[7] * xs[7])
            qsum = jnp.sum(q01 + q23)
            mean = ssum * (1.0 / HIDDEN)
            var = qsum * (1.0 / HIDDEN) - mean * mean
            veps = jnp.maximum(var, 0.0) + 1e-12
            v16 = lax.broadcast(veps, (LANES,))
            bits = plsc.bitcast(v16, jnp.int32)
            y = plsc.bitcast(jnp.int32(0x5F3759DF) - (bits >> 1), jnp.float32)
            nhalf = -0.5 * v16
            for _ in range(2):
                y = y * (1.5 + nhalf * y * y)
            for k in range(NREG):
                sl = pl.ds(k * LANES, LANES)
                g = gm[k] * y
                rows_v[i, sl] = (xs[k] - mean) * g + bt[k]

        pltpu.async_copy(rows_v, out.at[pl.ds((chunk0 + r) * CH, CH)], so[b])

    cpa0 = aux_copy(0, 0)
    cpa1 = aux_copy(1, 1)
    aux_copy(2, 2)
    cpa0.wait()
    gather_rows(0)
    cpa1.wait()
    gather_rows(1)

    def loop_body(p, carry):
        for b in range(NBUF):
            r = p * NBUF + b
            bn2 = (b + 2) % NBUF
            bn3 = (b + 3) % NBUF

            @pl.when(r + 3 < nchunks)
            def _():
                aux_copy(r + 3, bn3)

            @pl.when(r + 2 < nchunks)
            def _():
                pltpu.make_async_copy(aux.at[pl.ds(0, AUXW)],
                                      aux_bufs[bn2], sa[bn2]).wait()

                @pl.when(r >= 2)
                def _():
                    wait_out(bn2)

                gather_rows(bn2)

            @pl.when(r < nchunks)
            def _():
                wait_gather(b)
                compute_chunk(r, b)
        return carry

    lax.fori_loop(0, pl.cdiv(nchunks, NBUF), loop_body, 0)

    for b in range(NBUF):
        wait_out(b)


def kernel(input_ids, token_type_ids, word_emb, pos_emb, type_emb, gamma, beta):
    B, L = input_ids.shape
    ids = input_ids.astype(jnp.int32)
    tt = token_type_ids.astype(jnp.int32)
    nchunk = (B * L) // CH
    pidx = tt * SEQ + jnp.arange(L, dtype=jnp.int32)[None, :]
    aux = jnp.zeros((nchunk, AUXW), jnp.int32)
    aux = aux.at[:, IDS_A:IDS_A + CH].set(ids.reshape(nchunk, CH))
    aux = aux.at[:, PIDX:PIDX + CH].set(pidx.reshape(nchunk, CH))
    posb = jnp.concatenate(
        [pos_emb[:L] + type_emb[0][None, :], pos_emb[:L] + type_emb[1][None, :]])
    consts = jnp.stack([gamma, beta])

    mesh = plsc.VectorSubcoreMesh(core_axis_name="c", subcore_axis_name="s")
    run = functools.partial(
        pl.kernel,
        mesh=mesh,
        out_type=jax.ShapeDtypeStruct((B * L, HIDDEN), jnp.float32),
        compiler_params=pltpu.CompilerParams(needs_layout_passes=False),
        scratch_types=[
            pltpu.VMEM((AUXW,), jnp.int32),
            pltpu.VMEM((AUXW,), jnp.int32),
            pltpu.VMEM((AUXW,), jnp.int32),
            pltpu.VMEM((AUXW,), jnp.int32),
            pltpu.VMEM((CH, HIDDEN), jnp.float32),
            pltpu.VMEM((CH, HIDDEN), jnp.float32),
            pltpu.VMEM((CH, HIDDEN), jnp.float32),
            pltpu.VMEM((CH, HIDDEN), jnp.float32),
            pltpu.VMEM((2 * SEQ, HIDDEN), jnp.float32),
            pltpu.VMEM((2, HIDDEN), jnp.float32),
        ] + [pltpu.SemaphoreType.DMA] * 12,
    )(_body)
    out = run(aux.reshape(nchunk * AUXW), word_emb, posb, consts)
    return out.reshape(B, L, HIDDEN)
